# baseline (speedup 1.0000x reference)
import functools

import jax
import jax.numpy as jnp
from jax.experimental import pallas as pl
from jax.experimental.pallas import tpu as pltpu

B, HD = 4, 128
BF = jnp.float32


def _stage_body(adj_ref, src16_ref, hprev_ref,
                w1t_ref, b1_ref, w2t_ref, b2_ref,
                wiht_ref, bih_ref, whht_ref, bhh_ref,
                out_ref, out16_ref):
    adj16 = adj_ref[...].astype(BF)
    w1t = w1t_ref[...]
    w2t = w2t_ref[...]
    wiht = wiht_ref[...]
    whht = whht_ref[...]
    b1 = b1_ref[...]
    b2 = b2_ref[...]
    bih = bih_ref[...]
    bhh = bhh_ref[...]
    for b in range(B):
        agg = jnp.dot(adj16, src16_ref[b],
                      preferred_element_type=jnp.float32)
        h1 = jnp.dot(agg.astype(BF), w1t,
                     preferred_element_type=jnp.float32) + b1
        g = 0.5 * h1 * (1.0 + jax.lax.erf(h1 * 0.7071067811865476))
        x = jnp.dot(g.astype(BF), w2t,
                    preferred_element_type=jnp.float32) + b2
        hprev = hprev_ref[b]
        gi = jnp.dot(x.astype(BF), wiht,
                     preferred_element_type=jnp.float32) + bih
        gh = jnp.dot(hprev.astype(BF), whht,
                     preferred_element_type=jnp.float32) + bhh
        r = jax.nn.sigmoid(gi[:, :HD] + gh[:, :HD])
        z = jax.nn.sigmoid(gi[:, HD:2 * HD] + gh[:, HD:2 * HD])
        n = jnp.tanh(gi[:, 2 * HD:] + r * gh[:, 2 * HD:])
        new = (1.0 - z) * n + z * hprev
        out_ref[b] = new
        if out16_ref is not None:
            out16_ref[b] = new.astype(BF)


def _stage(adj, src16, hprev, w1t, b1, w2t, b2, wiht, bih, whht, bhh,
           tile_m, emit_bf16):
    m, k = adj.shape
    grid = (m // tile_m,)
    in_specs = [
        pl.BlockSpec((tile_m, k), lambda i: (i, 0)),
        pl.BlockSpec((B, k, HD), lambda i: (0, 0, 0)),
        pl.BlockSpec((B, tile_m, HD), lambda i: (0, i, 0)),
        pl.BlockSpec((HD, HD), lambda i: (0, 0)),
        pl.BlockSpec((1, HD), lambda i: (0, 0)),
        pl.BlockSpec((HD, HD), lambda i: (0, 0)),
        pl.BlockSpec((1, HD), lambda i: (0, 0)),
        pl.BlockSpec((HD, 3 * HD), lambda i: (0, 0)),
        pl.BlockSpec((1, 3 * HD), lambda i: (0, 0)),
        pl.BlockSpec((HD, 3 * HD), lambda i: (0, 0)),
        pl.BlockSpec((1, 3 * HD), lambda i: (0, 0)),
    ]
    out_shape = [jax.ShapeDtypeStruct((B, m, HD), jnp.float32)]
    out_specs = [pl.BlockSpec((B, tile_m, HD), lambda i: (0, i, 0))]
    if emit_bf16:
        out_shape.append(jax.ShapeDtypeStruct((B, m, HD), BF))
        out_specs.append(pl.BlockSpec((B, tile_m, HD), lambda i: (0, i, 0)))
        body = _stage_body
    else:
        body = functools.partial(_stage_body, out16_ref=None)
    outs = pl.pallas_call(
        body,
        grid=grid,
        in_specs=in_specs,
        out_specs=out_specs,
        out_shape=out_shape,
        compiler_params=pltpu.CompilerParams(
            dimension_semantics=("arbitrary",),
        ),
    )(adj, src16, hprev, w1t, b1, w2t, b2, wiht, bih, whht, bhh)
    return outs


def kernel(v_feats, c_feats, H, H_t, W1, b1, W2, b2,
           var_wih, var_whh, var_bih, var_bhh,
           chk_wih, chk_whh, chk_bih, chk_bhh):
    w1t = W1.T.astype(BF)
    w2t = W2.T.astype(BF)
    b1r = b1.reshape(1, HD)
    b2r = b2.reshape(1, HD)
    chk_wiht = chk_wih.T.astype(BF)
    chk_whht = chk_whh.T.astype(BF)
    var_wiht = var_wih.T.astype(BF)
    var_whht = var_whh.T.astype(BF)
    chk_bihr = chk_bih.reshape(1, 3 * HD)
    chk_bhhr = chk_bhh.reshape(1, 3 * HD)
    var_bihr = var_bih.reshape(1, 3 * HD)
    var_bhhr = var_bhh.reshape(1, 3 * HD)

    v16 = v_feats.astype(BF)
    c_new, c_new16 = _stage(H, v16, c_feats,
                            w1t, b1r, w2t, b2r,
                            chk_wiht, chk_bihr, chk_whht, chk_bhhr,
                            tile_m=200, emit_bf16=True)
    (v_new,) = _stage(H_t, c_new16, v_feats,
                      w1t, b1r, w2t, b2r,
                      var_wiht, var_bihr, var_whht, var_bhhr,
                      tile_m=400, emit_bf16=False)
    return (v_new, c_new)


# speedup vs baseline: 1.6543x; 1.6543x over previous
import jax
import jax.numpy as jnp
from jax.experimental import pallas as pl
from jax.experimental.pallas import tpu as pltpu

B, HD = 4, 128
N_ALL = B * HD


def _stage_body(adj_ref, src_ref, hprev_ref,
                w1t_ref, b1_ref, w2t_ref, b2_ref,
                wiht_ref, bih_ref, whht_ref, bhh_ref,
                out_ref, outT_ref):
    agg_all = jnp.dot(adj_ref[...], src_ref[...],
                      preferred_element_type=jnp.float32)
    w1t = w1t_ref[...]
    w2t = w2t_ref[...]
    wiht = wiht_ref[...]
    whht = whht_ref[...]
    b1 = b1_ref[...]
    b2 = b2_ref[...]
    bih = bih_ref[...]
    bhh = bhh_ref[...]
    for b in range(B):
        agg = agg_all[:, b * HD:(b + 1) * HD]
        h1 = jnp.dot(agg, w1t, preferred_element_type=jnp.float32) + b1
        g = 0.5 * h1 * (1.0 + jax.lax.erf(h1 * 0.7071067811865476))
        x = jnp.dot(g, w2t, preferred_element_type=jnp.float32) + b2
        hprev = hprev_ref[b]
        gi = jnp.dot(x, wiht, preferred_element_type=jnp.float32) + bih
        gh = jnp.dot(hprev, whht, preferred_element_type=jnp.float32) + bhh
        r = jax.nn.sigmoid(gi[:, :HD] + gh[:, :HD])
        z = jax.nn.sigmoid(gi[:, HD:2 * HD] + gh[:, HD:2 * HD])
        n = jnp.tanh(gi[:, 2 * HD:] + r * gh[:, 2 * HD:])
        new = (1.0 - z) * n + z * hprev
        out_ref[b] = new
        if outT_ref is not None:
            outT_ref[:, b * HD:(b + 1) * HD] = new


def _stage(adj, src, hprev, w1t, b1, w2t, b2, wiht, bih, whht, bhh,
           tile_m, emit_transposed):
    m, k = adj.shape
    grid = (m // tile_m,)
    in_specs = [
        pl.BlockSpec((tile_m, k), lambda i: (i, 0)),
        pl.BlockSpec((k, N_ALL), lambda i: (0, 0)),
        pl.BlockSpec((B, tile_m, HD), lambda i: (0, i, 0)),
        pl.BlockSpec((HD, HD), lambda i: (0, 0)),
        pl.BlockSpec((1, HD), lambda i: (0, 0)),
        pl.BlockSpec((HD, HD), lambda i: (0, 0)),
        pl.BlockSpec((1, HD), lambda i: (0, 0)),
        pl.BlockSpec((HD, 3 * HD), lambda i: (0, 0)),
        pl.BlockSpec((1, 3 * HD), lambda i: (0, 0)),
        pl.BlockSpec((HD, 3 * HD), lambda i: (0, 0)),
        pl.BlockSpec((1, 3 * HD), lambda i: (0, 0)),
    ]
    out_shape = [jax.ShapeDtypeStruct((B, m, HD), jnp.float32)]
    out_specs = [pl.BlockSpec((B, tile_m, HD), lambda i: (0, i, 0))]
    if emit_transposed:
        out_shape.append(jax.ShapeDtypeStruct((m, N_ALL), jnp.float32))
        out_specs.append(pl.BlockSpec((tile_m, N_ALL), lambda i: (i, 0)))
        body = _stage_body
    else:
        def body(*refs):
            _stage_body(*refs, outT_ref=None)
    return pl.pallas_call(
        body,
        grid=grid,
        in_specs=in_specs,
        out_specs=out_specs,
        out_shape=out_shape,
        compiler_params=pltpu.CompilerParams(
            dimension_semantics=("arbitrary",),
        ),
    )(adj, src, hprev, w1t, b1, w2t, b2, wiht, bih, whht, bhh)


def kernel(v_feats, c_feats, H, H_t, W1, b1, W2, b2,
           var_wih, var_whh, var_bih, var_bhh,
           chk_wih, chk_whh, chk_bih, chk_bhh):
    w1t = W1.T
    w2t = W2.T
    b1r = b1.reshape(1, HD)
    b2r = b2.reshape(1, HD)
    chk_wiht = chk_wih.T
    chk_whht = chk_whh.T
    var_wiht = var_wih.T
    var_whht = var_whh.T
    chk_bihr = chk_bih.reshape(1, 3 * HD)
    chk_bhhr = chk_bhh.reshape(1, 3 * HD)
    var_bihr = var_bih.reshape(1, 3 * HD)
    var_bhhr = var_bhh.reshape(1, 3 * HD)

    v_src = jnp.transpose(v_feats, (1, 0, 2)).reshape(-1, N_ALL)
    c_new, c_newT = _stage(H, v_src, c_feats,
                           w1t, b1r, w2t, b2r,
                           chk_wiht, chk_bihr, chk_whht, chk_bhhr,
                           tile_m=200, emit_transposed=True)
    (v_new,) = _stage(H_t, c_newT, v_feats,
                      w1t, b1r, w2t, b2r,
                      var_wiht, var_bihr, var_whht, var_bhhr,
                      tile_m=400, emit_transposed=False)
    return (v_new, c_new)


# speedup vs baseline: 1.6822x; 1.0169x over previous
import jax
import jax.numpy as jnp
from jax.experimental import pallas as pl
from jax.experimental.pallas import tpu as pltpu

B, HD = 4, 128
N_ALL = B * HD


def _stage_body(adj_ref, src_ref, hprev_ref,
                w1t_ref, b1_ref, w2t_ref, b2_ref,
                wiht_ref, bih_ref, whht_ref, bhh_ref,
                out_ref, outT_ref):
    agg_all = jnp.dot(adj_ref[...].astype(jnp.bfloat16), src_ref[...],
                      preferred_element_type=jnp.float32)
    w1t = w1t_ref[...]
    w2t = w2t_ref[...]
    wiht = wiht_ref[...]
    whht = whht_ref[...]
    b1 = b1_ref[...]
    b2 = b2_ref[...]
    bih = bih_ref[...]
    bhh = bhh_ref[...]
    for b in range(B):
        agg = agg_all[:, b * HD:(b + 1) * HD]
        h1 = jnp.dot(agg, w1t, preferred_element_type=jnp.float32) + b1
        g = 0.5 * h1 * (1.0 + jax.lax.erf(h1 * 0.7071067811865476))
        x = jnp.dot(g, w2t, preferred_element_type=jnp.float32) + b2
        hprev = hprev_ref[b]
        gi = jnp.dot(x, wiht, preferred_element_type=jnp.float32) + bih
        gh = jnp.dot(hprev, whht, preferred_element_type=jnp.float32) + bhh
        r = jax.nn.sigmoid(gi[:, :HD] + gh[:, :HD])
        z = jax.nn.sigmoid(gi[:, HD:2 * HD] + gh[:, HD:2 * HD])
        n = jnp.tanh(gi[:, 2 * HD:] + r * gh[:, 2 * HD:])
        new = (1.0 - z) * n + z * hprev
        out_ref[b] = new
        if outT_ref is not None:
            outT_ref[:, b * HD:(b + 1) * HD] = new.astype(jnp.bfloat16)


def _stage(adj, src, hprev, w1t, b1, w2t, b2, wiht, bih, whht, bhh,
           tile_m, emit_transposed):
    m, k = adj.shape
    grid = (m // tile_m,)
    in_specs = [
        pl.BlockSpec((tile_m, k), lambda i: (i, 0)),
        pl.BlockSpec((k, N_ALL), lambda i: (0, 0)),
        pl.BlockSpec((B, tile_m, HD), lambda i: (0, i, 0)),
        pl.BlockSpec((HD, HD), lambda i: (0, 0)),
        pl.BlockSpec((1, HD), lambda i: (0, 0)),
        pl.BlockSpec((HD, HD), lambda i: (0, 0)),
        pl.BlockSpec((1, HD), lambda i: (0, 0)),
        pl.BlockSpec((HD, 3 * HD), lambda i: (0, 0)),
        pl.BlockSpec((1, 3 * HD), lambda i: (0, 0)),
        pl.BlockSpec((HD, 3 * HD), lambda i: (0, 0)),
        pl.BlockSpec((1, 3 * HD), lambda i: (0, 0)),
    ]
    out_shape = [jax.ShapeDtypeStruct((B, m, HD), jnp.float32)]
    out_specs = [pl.BlockSpec((B, tile_m, HD), lambda i: (0, i, 0))]
    if emit_transposed:
        out_shape.append(jax.ShapeDtypeStruct((m, N_ALL), jnp.bfloat16))
        out_specs.append(pl.BlockSpec((tile_m, N_ALL), lambda i: (i, 0)))
        body = _stage_body
    else:
        def body(*refs):
            _stage_body(*refs, outT_ref=None)
    return pl.pallas_call(
        body,
        grid=grid,
        in_specs=in_specs,
        out_specs=out_specs,
        out_shape=out_shape,
        compiler_params=pltpu.CompilerParams(
            dimension_semantics=("arbitrary",),
        ),
    )(adj, src, hprev, w1t, b1, w2t, b2, wiht, bih, whht, bhh)


def kernel(v_feats, c_feats, H, H_t, W1, b1, W2, b2,
           var_wih, var_whh, var_bih, var_bhh,
           chk_wih, chk_whh, chk_bih, chk_bhh):
    w1t = W1.T
    w2t = W2.T
    b1r = b1.reshape(1, HD)
    b2r = b2.reshape(1, HD)
    chk_wiht = chk_wih.T
    chk_whht = chk_whh.T
    var_wiht = var_wih.T
    var_whht = var_whh.T
    chk_bihr = chk_bih.reshape(1, 3 * HD)
    chk_bhhr = chk_bhh.reshape(1, 3 * HD)
    var_bihr = var_bih.reshape(1, 3 * HD)
    var_bhhr = var_bhh.reshape(1, 3 * HD)

    v_src = jnp.transpose(v_feats, (1, 0, 2)).reshape(-1, N_ALL).astype(jnp.bfloat16)
    c_new, c_newT = _stage(H, v_src, c_feats,
                           w1t, b1r, w2t, b2r,
                           chk_wiht, chk_bihr, chk_whht, chk_bhhr,
                           tile_m=200, emit_transposed=True)
    (v_new,) = _stage(H_t, c_newT, v_feats,
                      w1t, b1r, w2t, b2r,
                      var_wiht, var_bihr, var_whht, var_bhhr,
                      tile_m=400, emit_transposed=False)
    return (v_new, c_new)


# speedup vs baseline: 1.6840x; 1.0011x over previous
import jax
import jax.numpy as jnp
from jax.experimental import pallas as pl
from jax.experimental.pallas import tpu as pltpu

B, HD = 4, 128
N_ALL = B * HD


def _stage_body(adj_ref, src_ref, hprev_ref,
                w1t_ref, b1_ref, w2t_ref, b2_ref,
                wiht_ref, bih_ref, whht_ref, bhh_ref,
                out_ref, outT_ref):
    agg_all = jnp.dot(adj_ref[...].astype(jnp.bfloat16), src_ref[...],
                      preferred_element_type=jnp.float32)
    w1t = w1t_ref[...]
    w2t = w2t_ref[...]
    wiht = wiht_ref[...]
    whht = whht_ref[...]
    b1 = b1_ref[...]
    b2 = b2_ref[...]
    bih = bih_ref[...]
    bhh = bhh_ref[...]
    for b in range(B):
        agg = agg_all[:, b * HD:(b + 1) * HD]
        h1 = jnp.dot(agg, w1t, preferred_element_type=jnp.float32) + b1
        g = 0.5 * h1 * (1.0 + jax.lax.erf(h1 * 0.7071067811865476))
        x = jnp.dot(g, w2t, preferred_element_type=jnp.float32) + b2
        hprev = hprev_ref[b]
        gi = jnp.dot(x, wiht, preferred_element_type=jnp.float32) + bih
        gh = jnp.dot(hprev, whht, preferred_element_type=jnp.float32) + bhh
        r = jax.nn.sigmoid(gi[:, :HD] + gh[:, :HD])
        z = jax.nn.sigmoid(gi[:, HD:2 * HD] + gh[:, HD:2 * HD])
        n = jnp.tanh(gi[:, 2 * HD:] + r * gh[:, 2 * HD:])
        new = (1.0 - z) * n + z * hprev
        out_ref[b] = new
        if outT_ref is not None:
            outT_ref[:, b * HD:(b + 1) * HD] = new.astype(jnp.bfloat16)


def _stage(adj, src, hprev, w1t, b1, w2t, b2, wiht, bih, whht, bhh,
           tile_m, emit_transposed):
    m, k = adj.shape
    grid = (m // tile_m,)
    in_specs = [
        pl.BlockSpec((tile_m, k), lambda i: (i, 0)),
        pl.BlockSpec((k, N_ALL), lambda i: (0, 0)),
        pl.BlockSpec((B, tile_m, HD), lambda i: (0, i, 0)),
        pl.BlockSpec((HD, HD), lambda i: (0, 0)),
        pl.BlockSpec((1, HD), lambda i: (0, 0)),
        pl.BlockSpec((HD, HD), lambda i: (0, 0)),
        pl.BlockSpec((1, HD), lambda i: (0, 0)),
        pl.BlockSpec((HD, 3 * HD), lambda i: (0, 0)),
        pl.BlockSpec((1, 3 * HD), lambda i: (0, 0)),
        pl.BlockSpec((HD, 3 * HD), lambda i: (0, 0)),
        pl.BlockSpec((1, 3 * HD), lambda i: (0, 0)),
    ]
    out_shape = [jax.ShapeDtypeStruct((B, m, HD), jnp.float32)]
    out_specs = [pl.BlockSpec((B, tile_m, HD), lambda i: (0, i, 0))]
    if emit_transposed:
        out_shape.append(jax.ShapeDtypeStruct((m, N_ALL), jnp.bfloat16))
        out_specs.append(pl.BlockSpec((tile_m, N_ALL), lambda i: (i, 0)))
        body = _stage_body
    else:
        def body(*refs):
            _stage_body(*refs, outT_ref=None)
    return pl.pallas_call(
        body,
        grid=grid,
        in_specs=in_specs,
        out_specs=out_specs,
        out_shape=out_shape,
        compiler_params=pltpu.CompilerParams(
            dimension_semantics=("parallel",),
        ),
    )(adj, src, hprev, w1t, b1, w2t, b2, wiht, bih, whht, bhh)


def kernel(v_feats, c_feats, H, H_t, W1, b1, W2, b2,
           var_wih, var_whh, var_bih, var_bhh,
           chk_wih, chk_whh, chk_bih, chk_bhh):
    w1t = W1.T
    w2t = W2.T
    b1r = b1.reshape(1, HD)
    b2r = b2.reshape(1, HD)
    chk_wiht = chk_wih.T
    chk_whht = chk_whh.T
    var_wiht = var_wih.T
    var_whht = var_whh.T
    chk_bihr = chk_bih.reshape(1, 3 * HD)
    chk_bhhr = chk_bhh.reshape(1, 3 * HD)
    var_bihr = var_bih.reshape(1, 3 * HD)
    var_bhhr = var_bhh.reshape(1, 3 * HD)

    v_src = jnp.transpose(v_feats, (1, 0, 2)).reshape(-1, N_ALL).astype(jnp.bfloat16)
    c_new, c_newT = _stage(H, v_src, c_feats,
                           w1t, b1r, w2t, b2r,
                           chk_wiht, chk_bihr, chk_whht, chk_bhhr,
                           tile_m=200, emit_transposed=True)
    (v_new,) = _stage(H_t, c_newT, v_feats,
                      w1t, b1r, w2t, b2r,
                      var_wiht, var_bihr, var_whht, var_bhhr,
                      tile_m=400, emit_transposed=False)
    return (v_new, c_new)


# speedup vs baseline: 1.8034x; 1.0709x over previous
import jax
import jax.numpy as jnp
from jax.experimental import pallas as pl
from jax.experimental.pallas import tpu as pltpu

B, HD = 4, 128
N_ALL = B * HD


def _stage_body(adj_ref, src_ref, hprev_ref,
                w1t_ref, b1_ref, w2t_ref, b2_ref,
                wiht_ref, bih_ref, whht_ref, bhh_ref,
                out_ref, outT_ref):
    agg_all = jnp.dot(adj_ref[...].astype(jnp.bfloat16), src_ref[...],
                      preferred_element_type=jnp.float32)
    w1t = w1t_ref[...]
    w2t = w2t_ref[...]
    wiht = wiht_ref[...]
    whht = whht_ref[...]
    b1 = b1_ref[...]
    b2 = b2_ref[...]
    bih = bih_ref[...]
    bhh = bhh_ref[...]
    for b in range(B):
        agg = agg_all[:, b * HD:(b + 1) * HD]
        h1 = jnp.dot(agg, w1t, preferred_element_type=jnp.float32) + b1
        g = 0.5 * h1 * (1.0 + jax.lax.erf(h1 * 0.7071067811865476))
        x = jnp.dot(g, w2t, preferred_element_type=jnp.float32) + b2
        hprev = hprev_ref[b]
        gi = jnp.dot(x, wiht, preferred_element_type=jnp.float32) + bih
        gh = jnp.dot(hprev, whht, preferred_element_type=jnp.float32) + bhh
        r = jax.nn.sigmoid(gi[:, :HD] + gh[:, :HD])
        z = jax.nn.sigmoid(gi[:, HD:2 * HD] + gh[:, HD:2 * HD])
        n = jnp.tanh(gi[:, 2 * HD:] + r * gh[:, 2 * HD:])
        new = (1.0 - z) * n + z * hprev
        out_ref[b] = new
        if outT_ref is not None:
            outT_ref[:, b * HD:(b + 1) * HD] = new.astype(jnp.bfloat16)


def _stage(adj, src, hprev, w1t, b1, w2t, b2, wiht, bih, whht, bhh,
           tile_m, emit_transposed):
    m, k = adj.shape
    grid = (pl.cdiv(m, tile_m),)
    in_specs = [
        pl.BlockSpec((tile_m, k), lambda i: (i, 0)),
        pl.BlockSpec((k, N_ALL), lambda i: (0, 0)),
        pl.BlockSpec((B, tile_m, HD), lambda i: (0, i, 0)),
        pl.BlockSpec((HD, HD), lambda i: (0, 0)),
        pl.BlockSpec((1, HD), lambda i: (0, 0)),
        pl.BlockSpec((HD, HD), lambda i: (0, 0)),
        pl.BlockSpec((1, HD), lambda i: (0, 0)),
        pl.BlockSpec((HD, 3 * HD), lambda i: (0, 0)),
        pl.BlockSpec((1, 3 * HD), lambda i: (0, 0)),
        pl.BlockSpec((HD, 3 * HD), lambda i: (0, 0)),
        pl.BlockSpec((1, 3 * HD), lambda i: (0, 0)),
    ]
    out_shape = [jax.ShapeDtypeStruct((B, m, HD), jnp.float32)]
    out_specs = [pl.BlockSpec((B, tile_m, HD), lambda i: (0, i, 0))]
    if emit_transposed:
        out_shape.append(jax.ShapeDtypeStruct((m, N_ALL), jnp.bfloat16))
        out_specs.append(pl.BlockSpec((tile_m, N_ALL), lambda i: (i, 0)))
        body = _stage_body
    else:
        def body(*refs):
            _stage_body(*refs, outT_ref=None)
    return pl.pallas_call(
        body,
        grid=grid,
        in_specs=in_specs,
        out_specs=out_specs,
        out_shape=out_shape,
        compiler_params=pltpu.CompilerParams(
            dimension_semantics=("parallel",),
        ),
    )(adj, src, hprev, w1t, b1, w2t, b2, wiht, bih, whht, bhh)


def kernel(v_feats, c_feats, H, H_t, W1, b1, W2, b2,
           var_wih, var_whh, var_bih, var_bhh,
           chk_wih, chk_whh, chk_bih, chk_bhh):
    w1t = W1.T
    w2t = W2.T
    b1r = b1.reshape(1, HD)
    b2r = b2.reshape(1, HD)
    chk_wiht = chk_wih.T
    chk_whht = chk_whh.T
    var_wiht = var_wih.T
    var_whht = var_whh.T
    chk_bihr = chk_bih.reshape(1, 3 * HD)
    chk_bhhr = chk_bhh.reshape(1, 3 * HD)
    var_bihr = var_bih.reshape(1, 3 * HD)
    var_bhhr = var_bhh.reshape(1, 3 * HD)

    v_src = jnp.transpose(v_feats, (1, 0, 2)).reshape(-1, N_ALL).astype(jnp.bfloat16)
    c_new, c_newT = _stage(H, v_src, c_feats,
                           w1t, b1r, w2t, b2r,
                           chk_wiht, chk_bihr, chk_whht, chk_bhhr,
                           tile_m=512, emit_transposed=True)
    (v_new,) = _stage(H_t, c_newT, v_feats,
                      w1t, b1r, w2t, b2r,
                      var_wiht, var_bihr, var_whht, var_bhhr,
                      tile_m=512, emit_transposed=False)
    return (v_new, c_new)


# speedup vs baseline: 1.8448x; 1.0229x over previous
import jax
import jax.numpy as jnp
from jax.experimental import pallas as pl
from jax.experimental.pallas import tpu as pltpu

B, HD = 4, 128
N_ALL = B * HD


def _stage_body(adj_ref, src_ref, hprev_ref,
                w1t_ref, b1_ref, w2t_ref, b2_ref,
                wiht_ref, bih_ref, whht_ref, bhh_ref,
                out_ref, outT_ref):
    agg_all = jnp.dot(adj_ref[...].astype(jnp.bfloat16), src_ref[...],
                      preferred_element_type=jnp.float32)
    w1t = w1t_ref[...]
    w2t = w2t_ref[...]
    wiht = wiht_ref[...]
    whht = whht_ref[...]
    b1 = b1_ref[...]
    b2 = b2_ref[...]
    bih = bih_ref[...]
    bhh = bhh_ref[...]
    for b in range(B):
        agg = agg_all[:, b * HD:(b + 1) * HD]
        h1 = jnp.dot(agg, w1t, preferred_element_type=jnp.float32) + b1
        g = 0.5 * h1 * (1.0 + jax.lax.erf(h1 * 0.7071067811865476))
        x = jnp.dot(g, w2t, preferred_element_type=jnp.float32) + b2
        hprev = hprev_ref[b]
        gi = jnp.dot(x, wiht, preferred_element_type=jnp.float32) + bih
        gh = jnp.dot(hprev, whht, preferred_element_type=jnp.float32) + bhh
        r = jax.nn.sigmoid(gi[:, :HD] + gh[:, :HD])
        z = jax.nn.sigmoid(gi[:, HD:2 * HD] + gh[:, HD:2 * HD])
        n = jnp.tanh(gi[:, 2 * HD:] + r * gh[:, 2 * HD:])
        new = (1.0 - z) * n + z * hprev
        out_ref[b] = new
        if outT_ref is not None:
            outT_ref[:, b * HD:(b + 1) * HD] = new.astype(jnp.bfloat16)


def _stage(adj, src, hprev, w1t, b1, w2t, b2, wiht, bih, whht, bhh,
           tile_m, emit_transposed):
    m, k = adj.shape
    grid = (pl.cdiv(m, tile_m),)
    in_specs = [
        pl.BlockSpec((tile_m, k), lambda i: (i, 0)),
        pl.BlockSpec((k, N_ALL), lambda i: (0, 0)),
        pl.BlockSpec((B, tile_m, HD), lambda i: (0, i, 0)),
        pl.BlockSpec((HD, HD), lambda i: (0, 0)),
        pl.BlockSpec((1, HD), lambda i: (0, 0)),
        pl.BlockSpec((HD, HD), lambda i: (0, 0)),
        pl.BlockSpec((1, HD), lambda i: (0, 0)),
        pl.BlockSpec((HD, 3 * HD), lambda i: (0, 0)),
        pl.BlockSpec((1, 3 * HD), lambda i: (0, 0)),
        pl.BlockSpec((HD, 3 * HD), lambda i: (0, 0)),
        pl.BlockSpec((1, 3 * HD), lambda i: (0, 0)),
    ]
    out_shape = [jax.ShapeDtypeStruct((B, m, HD), jnp.float32)]
    out_specs = [pl.BlockSpec((B, tile_m, HD), lambda i: (0, i, 0))]
    if emit_transposed:
        out_shape.append(jax.ShapeDtypeStruct((m, N_ALL), jnp.bfloat16))
        out_specs.append(pl.BlockSpec((tile_m, N_ALL), lambda i: (i, 0)))
        body = _stage_body
    else:
        def body(*refs):
            _stage_body(*refs, outT_ref=None)
    return pl.pallas_call(
        body,
        grid=grid,
        in_specs=in_specs,
        out_specs=out_specs,
        out_shape=out_shape,
        compiler_params=pltpu.CompilerParams(
            dimension_semantics=("parallel",),
        ),
    )(adj, src, hprev, w1t, b1, w2t, b2, wiht, bih, whht, bhh)


def kernel(v_feats, c_feats, H, H_t, W1, b1, W2, b2,
           var_wih, var_whh, var_bih, var_bhh,
           chk_wih, chk_whh, chk_bih, chk_bhh):
    w1t = W1.T
    w2t = W2.T
    b1r = b1.reshape(1, HD)
    b2r = b2.reshape(1, HD)
    chk_wiht = chk_wih.T
    chk_whht = chk_whh.T
    var_wiht = var_wih.T
    var_whht = var_whh.T
    chk_bihr = chk_bih.reshape(1, 3 * HD)
    chk_bhhr = chk_bhh.reshape(1, 3 * HD)
    var_bihr = var_bih.reshape(1, 3 * HD)
    var_bhhr = var_bhh.reshape(1, 3 * HD)

    v_src = jnp.transpose(v_feats, (1, 0, 2)).reshape(-1, N_ALL).astype(jnp.bfloat16)
    c_new, c_newT = _stage(H, v_src, c_feats,
                           w1t, b1r, w2t, b2r,
                           chk_wiht, chk_bihr, chk_whht, chk_bhhr,
                           tile_m=512, emit_transposed=True)
    (v_new,) = _stage(H_t, c_newT, v_feats,
                      w1t, b1r, w2t, b2r,
                      var_wiht, var_bihr, var_whht, var_bhhr,
                      tile_m=1024, emit_transposed=False)
    return (v_new, c_new)
